# baseline (speedup 1.0000x reference)
import functools

import jax
import jax.numpy as jnp
from jax import lax
from jax.experimental import pallas as pl
from jax.experimental.pallas import tpu as pltpu
from jax.experimental.pallas import tpu_sc as plsc

EMB_DIM = 128
NW = 32
CHUNK = 128
NSEG = EMB_DIM // 16


def _precompute_body(x_ref, ws_ref, bs_ref, wd_ref, bd_ref, hs_ref, hd_ref):
    xb = x_ref[...]
    hs_ref[...] = (
        jnp.dot(xb, ws_ref[...], preferred_element_type=jnp.float32) + bs_ref[...]
    ).astype(jnp.bfloat16)
    hd_ref[...] = (
        jnp.dot(xb, wd_ref[...], preferred_element_type=jnp.float32) + bd_ref[...]
    ).astype(jnp.bfloat16)


def _precompute(x, ws_t, bs, wd_t, bd):
    n, d = x.shape
    blk = 1000
    grid = n // blk
    return pl.pallas_call(
        _precompute_body,
        grid=(grid,),
        in_specs=[
            pl.BlockSpec((blk, d), lambda i: (i, 0)),
            pl.BlockSpec((d, d), lambda i: (0, 0)),
            pl.BlockSpec((1, d), lambda i: (0, 0)),
            pl.BlockSpec((d, d), lambda i: (0, 0)),
            pl.BlockSpec((1, d), lambda i: (0, 0)),
        ],
        out_specs=[
            pl.BlockSpec((blk, d), lambda i: (i, 0)),
            pl.BlockSpec((blk, d), lambda i: (i, 0)),
        ],
        out_shape=[
            jax.ShapeDtypeStruct((n, d), jnp.bfloat16),
            jax.ShapeDtypeStruct((n, d), jnp.bfloat16),
        ],
    )(x, ws_t, bs, wd_t, bd)


def _sc_score_body(hsrc, hdst, src_idx, dst_idx, wvec, out_hbm,
                   ixs_v, ixd_v, a_v, b_v, w_v, out_v,
                   sem_rows, sem_idx):
    n_chunks = src_idx.shape[1]
    per_w = n_chunks * CHUNK
    wid = lax.axis_index("s") * 2 + lax.axis_index("c")
    last = n_chunks - 1

    pltpu.sync_copy(wvec, w_v)
    b_out_s = w_v[pl.ds(EMB_DIM, 16)][0]
    iota16 = lax.broadcasted_iota(jnp.int32, (16,), 0)

    def idx_fetch(ci, buf):
        pltpu.async_copy(src_idx.at[wid, ci], ixs_v.at[buf], sem_idx.at[buf])
        pltpu.async_copy(dst_idx.at[wid, ci], ixd_v.at[buf], sem_idx.at[buf])

    def idx_wait(ci, buf):
        pltpu.make_async_copy(src_idx.at[wid, ci], ixs_v.at[buf],
                              sem_idx.at[buf]).wait()
        pltpu.make_async_copy(dst_idx.at[wid, ci], ixd_v.at[buf],
                              sem_idx.at[buf]).wait()

    def row_fetch(buf):
        pltpu.async_copy(hsrc.at[ixs_v.at[buf]], a_v.at[buf], sem_rows.at[buf])
        pltpu.async_copy(hdst.at[ixd_v.at[buf]], b_v.at[buf], sem_rows.at[buf])

    def row_wait(buf):
        pltpu.make_async_copy(hsrc.at[ixs_v.at[buf]], a_v.at[buf],
                              sem_rows.at[buf]).wait()
        pltpu.make_async_copy(hdst.at[ixd_v.at[buf]], b_v.at[buf],
                              sem_rows.at[buf]).wait()

    def compute(ci, buf):
        def group(g, _):
            ts = [jnp.zeros((16,), jnp.float32)] * 16
            for q in range(NSEG // 2):
                w_ev = w_v[pl.ds(q * 32, 16)]
                w_od = w_v[pl.ds(q * 32 + 16, 16)]
                for sub in range(16):
                    e = g * 16 + sub
                    s32 = (plsc.bitcast(a_v[buf, e, pl.ds(q * 16, 16)],
                                        jnp.bfloat16)
                           + plsc.bitcast(b_v[buf, e, pl.ds(q * 16, 16)],
                                          jnp.bfloat16))
                    ev, od = plsc.unpack(s32, format=plsc.PackFormat.INTERLEAVED)
                    ts[sub] = (ts[sub]
                               + jnp.maximum(ev, 0.0) * w_ev
                               + jnp.maximum(od, 0.0) * w_od)
            score = jnp.zeros((16,), jnp.float32)
            for sub in range(16):
                score = jnp.where(iota16 == sub, jnp.sum(ts[sub]), score)
            out_v[pl.ds(ci * CHUNK + g * 16, 16)] = score + b_out_s
            return 0

        lax.fori_loop(0, CHUNK // 16, group, 0)

    idx_fetch(0, 0)
    idx_fetch(1, 1)
    idx_fetch(2, 2)
    idx_wait(0, 0)
    row_fetch(0)
    idx_wait(1, 1)
    row_fetch(1)

    def chunk_step(c, cur, nxt2):
        @pl.when(c + 2 <= last)
        def _():
            idx_wait(c + 2, nxt2)
            row_fetch(nxt2)

        row_wait(cur)

        @pl.when(c + 3 <= last)
        def _():
            idx_fetch(c + 3, cur)

        compute(c, cur)

    def step3(i, _):
        c0 = i * 3
        chunk_step(c0, 0, 2)
        chunk_step(c0 + 1, 1, 0)
        chunk_step(c0 + 2, 2, 1)
        return 0

    lax.fori_loop(0, n_chunks // 3, step3, 0)
    pltpu.sync_copy(out_v, out_hbm.at[pl.ds(wid * per_w, per_w)])


def _sc_score(hs, hd, src_idx, dst_idx, wvec, n_edges_pad):
    per_w = n_edges_pad // NW
    n_chunks = per_w // CHUNK
    mesh = plsc.VectorSubcoreMesh(core_axis_name="c", subcore_axis_name="s")
    return pl.kernel(
        _sc_score_body,
        out_type=jax.ShapeDtypeStruct((n_edges_pad,), jnp.float32),
        mesh=mesh,
        scratch_types=[
            pltpu.VMEM((3, CHUNK), jnp.int32),
            pltpu.VMEM((3, CHUNK), jnp.int32),
            pltpu.VMEM((3, CHUNK, EMB_DIM // 2), jnp.int32),
            pltpu.VMEM((3, CHUNK, EMB_DIM // 2), jnp.int32),
            pltpu.VMEM((160,), jnp.float32),
            pltpu.VMEM((per_w,), jnp.float32),
            pltpu.SemaphoreType.DMA((3,)),
            pltpu.SemaphoreType.DMA((3,)),
        ],
        compiler_params=pltpu.CompilerParams(
            needs_layout_passes=False, use_tc_tiling_on_sc=False
        ),
    )(hs, hd, src_idx, dst_idx, wvec)


def kernel(x, pos_edge_index, neg_edge_index, W_src, b_src, W_dst, b_dst,
           W_out, b_out):
    e = pos_edge_index.shape[1]
    e2 = 2 * e
    n_chunks = -(-e2 // (NW * CHUNK))
    n_chunks += -n_chunks % 3
    per_w = n_chunks * CHUNK
    e2_pad = per_w * NW

    hs, hd = _precompute(
        x, W_src.T, b_src.reshape(1, -1), W_dst.T, b_dst.reshape(1, -1)
    )
    n = hs.shape[0]
    hs = jax.lax.bitcast_convert_type(
        hs.reshape(n, EMB_DIM // 2, 2), jnp.int32)
    hd = jax.lax.bitcast_convert_type(
        hd.reshape(n, EMB_DIM // 2, 2), jnp.int32)

    pad = e2_pad - e2
    src = jnp.concatenate(
        [pos_edge_index[0], neg_edge_index[0],
         jnp.zeros((pad,), pos_edge_index.dtype)]
    ).astype(jnp.int32).reshape(NW, n_chunks, CHUNK)
    dst = jnp.concatenate(
        [pos_edge_index[1], neg_edge_index[1],
         jnp.zeros((pad,), pos_edge_index.dtype)]
    ).astype(jnp.int32).reshape(NW, n_chunks, CHUNK)
    w_perm = (
        W_out.reshape(EMB_DIM // 32, 16, 2)
        .transpose(0, 2, 1)
        .reshape(-1)
    )
    wvec = jnp.concatenate(
        [w_perm, b_out.reshape(-1),
         jnp.zeros((160 - EMB_DIM - 1,), jnp.float32)]
    )

    out = _sc_score(hs, hd, src, dst, wvec, e2_pad)
    return out[:e].reshape(e, 1), out[e:e2].reshape(e, 1)


# speedup vs baseline: 1.0652x; 1.0652x over previous
import functools

import jax
import jax.numpy as jnp
from jax import lax
from jax.experimental import pallas as pl
from jax.experimental.pallas import tpu as pltpu
from jax.experimental.pallas import tpu_sc as plsc

EMB_DIM = 128
NW = 32
CHUNK = 128
NSEG = EMB_DIM // 16


def _precompute_body(x_ref, ws_ref, bs_ref, wd_ref, bd_ref, hs_ref, hd_ref):
    xb = x_ref[...]
    hs_ref[...] = (
        jnp.dot(xb, ws_ref[...], preferred_element_type=jnp.float32) + bs_ref[...]
    ).astype(jnp.bfloat16)
    hd_ref[...] = (
        jnp.dot(xb, wd_ref[...], preferred_element_type=jnp.float32) + bd_ref[...]
    ).astype(jnp.bfloat16)


def _precompute(x, ws_t, bs, wd_t, bd):
    n, d = x.shape
    blk = 1000
    grid = n // blk
    return pl.pallas_call(
        _precompute_body,
        grid=(grid,),
        in_specs=[
            pl.BlockSpec((blk, d), lambda i: (i, 0)),
            pl.BlockSpec((d, d), lambda i: (0, 0)),
            pl.BlockSpec((1, d), lambda i: (0, 0)),
            pl.BlockSpec((d, d), lambda i: (0, 0)),
            pl.BlockSpec((1, d), lambda i: (0, 0)),
        ],
        out_specs=[
            pl.BlockSpec((blk, d), lambda i: (i, 0)),
            pl.BlockSpec((blk, d), lambda i: (i, 0)),
        ],
        out_shape=[
            jax.ShapeDtypeStruct((n, d), jnp.bfloat16),
            jax.ShapeDtypeStruct((n, d), jnp.bfloat16),
        ],
    )(x, ws_t, bs, wd_t, bd)


def _sc_score_body(hsrc, hdst, src_idx, dst_idx, wvec, out_hbm,
                   ixs_v, ixd_v, a_v, b_v, w_v, out_v,
                   sem_rows, sem_idx):
    n_chunks = src_idx.shape[1]
    per_w = n_chunks * CHUNK
    wid = lax.axis_index("s") * 2 + lax.axis_index("c")
    last = n_chunks - 1

    pltpu.sync_copy(wvec, w_v)
    w_regs = [w_v[pl.ds(j * 16, 16)] for j in range(NSEG)]
    b_out_s = w_v[pl.ds(EMB_DIM, 16)][0]
    iota16 = lax.broadcasted_iota(jnp.int32, (16,), 0)

    def idx_fetch(ci, buf):
        pltpu.async_copy(src_idx.at[wid, ci], ixs_v.at[buf], sem_idx.at[buf])
        pltpu.async_copy(dst_idx.at[wid, ci], ixd_v.at[buf], sem_idx.at[buf])

    def idx_wait(ci, buf):
        pltpu.make_async_copy(src_idx.at[wid, ci], ixs_v.at[buf],
                              sem_idx.at[buf]).wait()
        pltpu.make_async_copy(dst_idx.at[wid, ci], ixd_v.at[buf],
                              sem_idx.at[buf]).wait()

    def row_fetch(buf):
        pltpu.async_copy(hsrc.at[ixs_v.at[buf]], a_v.at[buf], sem_rows.at[buf])
        pltpu.async_copy(hdst.at[ixd_v.at[buf]], b_v.at[buf], sem_rows.at[buf])

    def row_wait(buf):
        pltpu.make_async_copy(hsrc.at[ixs_v.at[buf]], a_v.at[buf],
                              sem_rows.at[buf]).wait()
        pltpu.make_async_copy(hdst.at[ixd_v.at[buf]], b_v.at[buf],
                              sem_rows.at[buf]).wait()

    def compute(ci, buf):
        def group(g, _):
            score = jnp.zeros((16,), jnp.float32)
            for half in range(2):
                ts = [jnp.zeros((16,), jnp.float32)] * 8
                for q in range(NSEG // 2):
                    w_ev = w_v[pl.ds(q * 32, 16)]
                    w_od = w_v[pl.ds(q * 32 + 16, 16)]
                    for sub in range(8):
                        e = g * 16 + half * 8 + sub
                        s32 = (plsc.bitcast(a_v[buf, e, pl.ds(q * 16, 16)],
                                            jnp.bfloat16)
                               + plsc.bitcast(b_v[buf, e, pl.ds(q * 16, 16)],
                                              jnp.bfloat16))
                        ev, od = plsc.unpack(s32,
                                             format=plsc.PackFormat.INTERLEAVED)
                        ts[sub] = (ts[sub]
                                   + jnp.maximum(ev, 0.0) * w_ev
                                   + jnp.maximum(od, 0.0) * w_od)
                for sub in range(8):
                    score = jnp.where(iota16 == half * 8 + sub,
                                      jnp.sum(ts[sub]), score)
            out_v[pl.ds(ci * CHUNK + g * 16, 16)] = score + b_out_s
            return 0

        lax.fori_loop(0, CHUNK // 16, group, 0)

    idx_fetch(0, 0)
    idx_fetch(1, 1)
    idx_wait(0, 0)
    row_fetch(0)

    def chunk_step(c, buf, obuf):
        idx_wait(jnp.minimum(c + 1, last), obuf)
        row_fetch(obuf)
        row_wait(buf)
        idx_fetch(jnp.minimum(c + 2, last), buf)
        compute(c, buf)

    def step2(i, _):
        c0 = i * 2
        chunk_step(c0, 0, 1)
        chunk_step(c0 + 1, 1, 0)
        return 0

    lax.fori_loop(0, n_chunks // 2, step2, 0)
    row_wait(0)
    idx_wait(last, 1)
    pltpu.sync_copy(out_v, out_hbm.at[pl.ds(wid * per_w, per_w)])


def _sc_score(hs, hd, src_idx, dst_idx, wvec, n_edges_pad):
    per_w = n_edges_pad // NW
    n_chunks = per_w // CHUNK
    mesh = plsc.VectorSubcoreMesh(core_axis_name="c", subcore_axis_name="s")
    return pl.kernel(
        _sc_score_body,
        out_type=jax.ShapeDtypeStruct((n_edges_pad,), jnp.float32),
        mesh=mesh,
        scratch_types=[
            pltpu.VMEM((2, CHUNK), jnp.int32),
            pltpu.VMEM((2, CHUNK), jnp.int32),
            pltpu.VMEM((2, CHUNK, EMB_DIM // 2), jnp.int32),
            pltpu.VMEM((2, CHUNK, EMB_DIM // 2), jnp.int32),
            pltpu.VMEM((160,), jnp.float32),
            pltpu.VMEM((per_w,), jnp.float32),
            pltpu.SemaphoreType.DMA((2,)),
            pltpu.SemaphoreType.DMA((2,)),
        ],
        compiler_params=pltpu.CompilerParams(
            needs_layout_passes=False, use_tc_tiling_on_sc=False
        ),
    )(hs, hd, src_idx, dst_idx, wvec)


def kernel(x, pos_edge_index, neg_edge_index, W_src, b_src, W_dst, b_dst,
           W_out, b_out):
    e = pos_edge_index.shape[1]
    e2 = 2 * e
    n_chunks = -(-e2 // (NW * CHUNK))
    n_chunks += n_chunks % 2
    per_w = n_chunks * CHUNK
    e2_pad = per_w * NW

    hs, hd = _precompute(
        x, W_src.T, b_src.reshape(1, -1), W_dst.T, b_dst.reshape(1, -1)
    )
    n = hs.shape[0]
    hs = jax.lax.bitcast_convert_type(
        hs.reshape(n, EMB_DIM // 2, 2), jnp.int32)
    hd = jax.lax.bitcast_convert_type(
        hd.reshape(n, EMB_DIM // 2, 2), jnp.int32)

    pad = e2_pad - e2
    src = jnp.concatenate(
        [pos_edge_index[0], neg_edge_index[0],
         jnp.zeros((pad,), pos_edge_index.dtype)]
    ).astype(jnp.int32).reshape(NW, n_chunks, CHUNK)
    dst = jnp.concatenate(
        [pos_edge_index[1], neg_edge_index[1],
         jnp.zeros((pad,), pos_edge_index.dtype)]
    ).astype(jnp.int32).reshape(NW, n_chunks, CHUNK)
    w_perm = (
        W_out.reshape(EMB_DIM // 32, 16, 2)
        .transpose(0, 2, 1)
        .reshape(-1)
    )
    wvec = jnp.concatenate(
        [w_perm, b_out.reshape(-1),
         jnp.zeros((160 - EMB_DIM - 1,), jnp.float32)]
    )

    out = _sc_score(hs, hd, src, dst, wvec, e2_pad)
    return out[:e].reshape(e, 1), out[e:e2].reshape(e, 1)
